# baseline (speedup 1.0000x reference)
import functools

import jax
import jax.numpy as jnp
from jax import lax
from jax.experimental import pallas as pl
from jax.experimental.pallas import tpu as pltpu
from jax.experimental.pallas import tpu_sc as plsc

N_CAT = 26
VOCAB = 100000
EMB = 32
B = 1024
S = 50
HID = N_CAT * EMB
OUT = 128

TCH = 1024
VP = 98 * TCH

NC = 2
NS = 16
NW = NC * NS
BBLK = 8
SWEEPS = B // (NW * BBLK)


@functools.lru_cache(maxsize=8)
def _make_repack(nf):
    def body(*refs):
        x_refs = refs[:nf]
        o_refs = refs[nf:]
        for t in range(nf):
            xt = lax.transpose(x_refs[t][...], (1, 0))
            o_refs[t][...] = jnp.concatenate([xt, xt, xt, xt], axis=1)

    return pl.pallas_call(
        body,
        grid=(VP // TCH,),
        in_specs=[pl.BlockSpec((EMB, TCH), lambda i: (0, i))] * nf,
        out_specs=[pl.BlockSpec((TCH, 128), lambda i: (i, 0))] * nf,
        out_shape=[jax.ShapeDtypeStruct((VP, 128), jnp.float32)] * nf,
    )


def _mish16(x):
    e = jnp.exp(x)
    y = 1.0 + e
    d = y * y + 1.0
    r = x / d
    return x - (r + r)


def _sc_body(nf, *refs):
    cats_ref = refs[0]
    emb_refs = refs[1:1 + nf]
    out_ref = refs[1 + nf]
    (idx_v0, idx_v1, rows_v0, rows_v1, pooled_v,
     gsem0, gsem1, wsem) = refs[2 + nf:]
    idx_bufs = (idx_v0, idx_v1)
    row_bufs = (rows_v0, rows_v1)
    gsems = (gsem0, gsem1)
    wid = lax.axis_index("s") * NC + lax.axis_index("c")
    tasks = [(f, sw) for f in range(nf) for sw in range(SWEEPS)]

    def task_base(sw):
        return wid * (BBLK * SWEEPS) + sw * BBLK

    def stage(k):
        f, sw = tasks[k]
        buf = k % 2
        idx_v, rows_v, gsem = idx_bufs[buf], row_bufs[buf], gsems[buf]
        b0 = task_base(sw)
        pltpu.sync_copy(cats_ref.at[f, pl.ds(b0, BBLK)], idx_v)
        emb_ref = emb_refs[f]

        def fire(i, carry):
            pltpu.async_copy(emb_ref.at[idx_v.at[i]], rows_v.at[i], gsem)
            return carry

        lax.fori_loop(0, BBLK, fire, 0)

    stage(0)
    for k, (f, sw) in enumerate(tasks):
        buf = k % 2
        rows_v, gsem = row_bufs[buf], gsems[buf]
        emb_ref = emb_refs[f]
        b0 = task_base(sw)
        if k + 1 < len(tasks):
            stage(k + 1)

        def drain(i, carry, emb_ref=emb_ref, rows_v=rows_v, gsem=gsem):
            pltpu.make_async_copy(emb_ref.at[pl.ds(0, S)], rows_v.at[i],
                                  gsem).wait()
            return carry

        lax.fori_loop(0, BBLK, drain, 0)

        def pool_row(i, carry, rows_v=rows_v):
            def seq_step(kk, acc):
                a0, a1 = acc
                for u in range(5):
                    s = kk * 5 + u
                    x0 = rows_v[i, s, pl.ds(0, 16)]
                    x1 = rows_v[i, s, pl.ds(16, 16)]
                    a0 = a0 + _mish16(x0)
                    a1 = a1 + _mish16(x1)
                return a0, a1

            z = jnp.zeros((16,), jnp.float32)
            a0, a1 = lax.fori_loop(0, S // 5, seq_step, (z, z))
            pooled_v[i, pl.ds(0, 16)] = a0 * (1.0 / S)
            pooled_v[i, pl.ds(16, 16)] = a1 * (1.0 / S)
            return carry

        lax.fori_loop(0, BBLK, pool_row, 0)
        pltpu.async_copy(pooled_v, out_ref.at[f, pl.ds(b0, BBLK)],
                         wsem).wait()


@functools.lru_cache(maxsize=8)
def _make_sc_pool(nf):
    return pl.kernel(
        functools.partial(_sc_body, nf),
        out_type=jax.ShapeDtypeStruct((nf, B, EMB), jnp.float32),
        mesh=plsc.VectorSubcoreMesh(core_axis_name="c", subcore_axis_name="s",
                                    num_cores=NC, num_subcores=NS),
        scratch_types=[
            pltpu.VMEM((BBLK, S), jnp.int32),
            pltpu.VMEM((BBLK, S), jnp.int32),
            pltpu.VMEM((BBLK, S, 128), jnp.float32),
            pltpu.VMEM((BBLK, S, 128), jnp.float32),
            pltpu.VMEM((BBLK, EMB), jnp.float32),
            pltpu.SemaphoreType.DMA,
            pltpu.SemaphoreType.DMA,
            pltpu.SemaphoreType.DMA,
        ],
        compiler_params=pltpu.CompilerParams(use_tc_tiling_on_sc=False),
    )


BC = 64


def _head_body(x_ref, w_ref, b_ref, o_ref):
    acc = b_ref[...].astype(jnp.float32) * jnp.ones((BC, 1), jnp.float32)
    for f in range(N_CAT):
        acc = acc + jax.lax.dot(x_ref[f], w_ref[f],
                                precision=lax.Precision.HIGHEST)
    o_ref[...] = acc


_head = pl.pallas_call(
    _head_body,
    grid=(B // BC,),
    in_specs=[
        pl.BlockSpec((N_CAT, BC, EMB), lambda i: (0, i, 0)),
        pl.BlockSpec((N_CAT, EMB, OUT), lambda i: (0, 0, 0)),
        pl.BlockSpec((1, OUT), lambda i: (0, 0)),
    ],
    out_specs=pl.BlockSpec((BC, OUT), lambda i: (i, 0)),
    out_shape=jax.ShapeDtypeStruct((B, OUT), jnp.float32),
)


def kernel(cat_0, cat_1, cat_2, cat_3, cat_4, cat_5, cat_6, cat_7, cat_8,
           cat_9, cat_10, cat_11, cat_12, cat_13, cat_14, cat_15, cat_16,
           cat_17, cat_18, cat_19, cat_20, cat_21, cat_22, cat_23, cat_24,
           cat_25,
           emb_0, emb_1, emb_2, emb_3, emb_4, emb_5, emb_6, emb_7, emb_8,
           emb_9, emb_10, emb_11, emb_12, emb_13, emb_14, emb_15, emb_16,
           emb_17, emb_18, emb_19, emb_20, emb_21, emb_22, emb_23, emb_24,
           emb_25,
           W, b):
    cats = [cat_0, cat_1, cat_2, cat_3, cat_4, cat_5, cat_6, cat_7, cat_8,
            cat_9, cat_10, cat_11, cat_12, cat_13, cat_14, cat_15, cat_16,
            cat_17, cat_18, cat_19, cat_20, cat_21, cat_22, cat_23, cat_24,
            cat_25]
    embs = [emb_0, emb_1, emb_2, emb_3, emb_4, emb_5, emb_6, emb_7, emb_8,
            emb_9, emb_10, emb_11, emb_12, emb_13, emb_14, emb_15, emb_16,
            emb_17, emb_18, emb_19, emb_20, emb_21, emb_22, emb_23, emb_24,
            emb_25]
    groups = [7, 7, 6, 4, 2]
    pieces = []
    f0 = 0
    for nf in groups:
        cats_g = jnp.stack(cats[f0:f0 + nf])
        embs_t = [jnp.transpose(e) for e in embs[f0:f0 + nf]]
        embs_rm = _make_repack(nf)(*embs_t)
        pieces.append(_make_sc_pool(nf)(cats_g, *embs_rm))
        f0 += nf
    pooled = jnp.concatenate(pieces, axis=0)
    w3 = W.reshape(N_CAT, EMB, OUT)
    b2 = b.reshape(1, OUT)
    return _head(pooled, w3, b2)


# speedup vs baseline: 1.1623x; 1.1623x over previous
import functools

import jax
import jax.numpy as jnp
from jax import lax
from jax.experimental import pallas as pl
from jax.experimental.pallas import tpu as pltpu
from jax.experimental.pallas import tpu_sc as plsc

N_CAT = 26
VOCAB = 100000
EMB = 32
B = 1024
S = 50
HID = N_CAT * EMB
OUT = 128

TCH = 2048
VP = 49 * TCH

NC = 2
NS = 16
NW = NC * NS
BBLK = 8
SWEEPS = B // (NW * BBLK)


@functools.lru_cache(maxsize=8)
def _make_repack(nf):
    def body(*refs):
        x_refs = refs[:nf]
        o_refs = refs[nf:]
        rowk = lax.broadcasted_iota(jnp.int32, (EMB, 128), 0)
        coll = lax.broadcasted_iota(jnp.int32, (EMB, 128), 1)
        eyed = jnp.where(coll % EMB == rowk, 1.0, 0.0).astype(jnp.float32)
        for t in range(nf):
            o_refs[t][...] = lax.dot_general(
                x_refs[t][...], eyed, (((0,), (0,)), ((), ())),
                precision=lax.Precision.HIGHEST)

    return pl.pallas_call(
        body,
        grid=(VP // TCH,),
        in_specs=[pl.BlockSpec((EMB, TCH), lambda i: (0, i))] * nf,
        out_specs=[pl.BlockSpec((TCH, 128), lambda i: (i, 0))] * nf,
        out_shape=[jax.ShapeDtypeStruct((VP, 128), jnp.float32)] * nf,
    )


def _mish16(x):
    e = jnp.exp(x)
    y = 1.0 + e
    d = y * y + 1.0
    r = x / d
    return x - (r + r)


def _sc_body(nf, *refs):
    cats_ref = refs[0]
    emb_refs = refs[1:1 + nf]
    out_ref = refs[1 + nf]
    (idx_v0, idx_v1, rows_v0, rows_v1, pooled_v,
     gsem0, gsem1, wsem) = refs[2 + nf:]
    idx_bufs = (idx_v0, idx_v1)
    row_bufs = (rows_v0, rows_v1)
    gsems = (gsem0, gsem1)
    wid = lax.axis_index("s") * NC + lax.axis_index("c")
    tasks = [(f, sw) for f in range(nf) for sw in range(SWEEPS)]

    def task_base(sw):
        return wid * (BBLK * SWEEPS) + sw * BBLK

    def stage(k):
        f, sw = tasks[k]
        buf = k % 2
        idx_v, rows_v, gsem = idx_bufs[buf], row_bufs[buf], gsems[buf]
        b0 = task_base(sw)
        pltpu.sync_copy(cats_ref.at[f, pl.ds(b0, BBLK)], idx_v)
        emb_ref = emb_refs[f]

        def fire(i, carry):
            pltpu.async_copy(emb_ref.at[idx_v.at[i]], rows_v.at[i], gsem)
            return carry

        lax.fori_loop(0, BBLK, fire, 0)

    stage(0)
    for k, (f, sw) in enumerate(tasks):
        buf = k % 2
        rows_v, gsem = row_bufs[buf], gsems[buf]
        emb_ref = emb_refs[f]
        b0 = task_base(sw)
        if k + 1 < len(tasks):
            stage(k + 1)

        def drain(i, carry, emb_ref=emb_ref, rows_v=rows_v, gsem=gsem):
            pltpu.make_async_copy(emb_ref.at[pl.ds(0, S)], rows_v.at[i],
                                  gsem).wait()
            return carry

        lax.fori_loop(0, BBLK, drain, 0)

        def pool_row(i, carry, rows_v=rows_v):
            def seq_step(kk, acc):
                a0, a1 = acc
                for u in range(5):
                    s = kk * 5 + u
                    x0 = rows_v[i, s, pl.ds(0, 16)]
                    x1 = rows_v[i, s, pl.ds(16, 16)]
                    a0 = a0 + _mish16(x0)
                    a1 = a1 + _mish16(x1)
                return a0, a1

            z = jnp.zeros((16,), jnp.float32)
            a0, a1 = lax.fori_loop(0, S // 5, seq_step, (z, z))
            pooled_v[i, pl.ds(0, 16)] = a0 * (1.0 / S)
            pooled_v[i, pl.ds(16, 16)] = a1 * (1.0 / S)
            return carry

        lax.fori_loop(0, BBLK, pool_row, 0)
        pltpu.async_copy(pooled_v, out_ref.at[f, pl.ds(b0, BBLK)],
                         wsem).wait()


@functools.lru_cache(maxsize=8)
def _make_sc_pool(nf):
    return pl.kernel(
        functools.partial(_sc_body, nf),
        out_type=jax.ShapeDtypeStruct((nf, B, EMB), jnp.float32),
        mesh=plsc.VectorSubcoreMesh(core_axis_name="c", subcore_axis_name="s",
                                    num_cores=NC, num_subcores=NS),
        scratch_types=[
            pltpu.VMEM((BBLK, S), jnp.int32),
            pltpu.VMEM((BBLK, S), jnp.int32),
            pltpu.VMEM((BBLK, S, 128), jnp.float32),
            pltpu.VMEM((BBLK, S, 128), jnp.float32),
            pltpu.VMEM((BBLK, EMB), jnp.float32),
            pltpu.SemaphoreType.DMA,
            pltpu.SemaphoreType.DMA,
            pltpu.SemaphoreType.DMA,
        ],
        compiler_params=pltpu.CompilerParams(use_tc_tiling_on_sc=False),
    )


BC = 64


def _head_body(x_ref, w_ref, b_ref, o_ref):
    acc = b_ref[...].astype(jnp.float32) * jnp.ones((BC, 1), jnp.float32)
    for f in range(N_CAT):
        acc = acc + jax.lax.dot(x_ref[f], w_ref[f],
                                precision=lax.Precision.HIGHEST)
    o_ref[...] = acc


_head = pl.pallas_call(
    _head_body,
    grid=(B // BC,),
    in_specs=[
        pl.BlockSpec((N_CAT, BC, EMB), lambda i: (0, i, 0)),
        pl.BlockSpec((N_CAT, EMB, OUT), lambda i: (0, 0, 0)),
        pl.BlockSpec((1, OUT), lambda i: (0, 0)),
    ],
    out_specs=pl.BlockSpec((BC, OUT), lambda i: (i, 0)),
    out_shape=jax.ShapeDtypeStruct((B, OUT), jnp.float32),
)


def kernel(cat_0, cat_1, cat_2, cat_3, cat_4, cat_5, cat_6, cat_7, cat_8,
           cat_9, cat_10, cat_11, cat_12, cat_13, cat_14, cat_15, cat_16,
           cat_17, cat_18, cat_19, cat_20, cat_21, cat_22, cat_23, cat_24,
           cat_25,
           emb_0, emb_1, emb_2, emb_3, emb_4, emb_5, emb_6, emb_7, emb_8,
           emb_9, emb_10, emb_11, emb_12, emb_13, emb_14, emb_15, emb_16,
           emb_17, emb_18, emb_19, emb_20, emb_21, emb_22, emb_23, emb_24,
           emb_25,
           W, b):
    cats = [cat_0, cat_1, cat_2, cat_3, cat_4, cat_5, cat_6, cat_7, cat_8,
            cat_9, cat_10, cat_11, cat_12, cat_13, cat_14, cat_15, cat_16,
            cat_17, cat_18, cat_19, cat_20, cat_21, cat_22, cat_23, cat_24,
            cat_25]
    embs = [emb_0, emb_1, emb_2, emb_3, emb_4, emb_5, emb_6, emb_7, emb_8,
            emb_9, emb_10, emb_11, emb_12, emb_13, emb_14, emb_15, emb_16,
            emb_17, emb_18, emb_19, emb_20, emb_21, emb_22, emb_23, emb_24,
            emb_25]
    groups = [7, 7, 6, 4, 2]
    pieces = []
    f0 = 0
    for nf in groups:
        cats_g = jnp.stack(cats[f0:f0 + nf])
        embs_t = [jnp.transpose(e) for e in embs[f0:f0 + nf]]
        embs_rm = _make_repack(nf)(*embs_t)
        pieces.append(_make_sc_pool(nf)(cats_g, *embs_rm))
        f0 += nf
    pooled = jnp.concatenate(pieces, axis=0)
    w3 = W.reshape(N_CAT, EMB, OUT)
    b2 = b.reshape(1, OUT)
    return _head(pooled, w3, b2)


# speedup vs baseline: 2.1046x; 1.8108x over previous
import functools

import jax
import jax.numpy as jnp
from jax import lax
from jax.experimental import pallas as pl
from jax.experimental.pallas import tpu as pltpu
from jax.experimental.pallas import tpu_sc as plsc

N_CAT = 26
VOCAB = 100000
EMB = 32
B = 1024
S = 50
HID = N_CAT * EMB
OUT = 128

TCH = 1792
VP = 14 * TCH

NC = 2
NS = 16
NW = NC * NS
BBLK = 8
SWEEPS = B // (NW * BBLK)


@functools.lru_cache(maxsize=8)
def _make_repack(nf):
    def body(*refs):
        x_refs = refs[:4 * nf]
        o_refs = refs[4 * nf:]
        rowk = lax.broadcasted_iota(jnp.int32, (128, 128), 0)
        coll = lax.broadcasted_iota(jnp.int32, (128, 128), 1)
        eye = jnp.where(coll == rowk, 1.0, 0.0).astype(jnp.float32)
        for t in range(nf):
            xcat = jnp.concatenate(
                [x_refs[4 * t + q][...] for q in range(4)], axis=0)
            o_refs[t][...] = lax.dot_general(
                xcat, eye, (((0,), (0,)), ((), ())),
                precision=lax.Precision.HIGHEST)

    in_specs = []
    for _ in range(nf):
        for q in range(4):
            in_specs.append(
                pl.BlockSpec((EMB, TCH),
                             lambda i, q=q: (0, q * (VP // TCH) + i)))
    return pl.pallas_call(
        body,
        grid=(VP // TCH,),
        in_specs=in_specs,
        out_specs=[pl.BlockSpec((TCH, 128), lambda i: (i, 0))] * nf,
        out_shape=[jax.ShapeDtypeStruct((VP, 128), jnp.float32)] * nf,
    )


def _mish16(x):
    e = jnp.exp(x)
    y = 1.0 + e
    d = y * y + 1.0
    r = x / d
    return x - (r + r)


def _sc_body(nf, *refs):
    cats_ref = refs[0]
    offs_ref = refs[1]
    emb_refs = refs[2:2 + nf]
    out_ref = refs[2 + nf]
    (idx_v0, idx_v1, off_v0, off_v1, rows_v0, rows_v1, pooled_v,
     gsem0, gsem1, wsem) = refs[3 + nf:]
    idx_bufs = (idx_v0, idx_v1)
    off_bufs = (off_v0, off_v1)
    row_bufs = (rows_v0, rows_v1)
    gsems = (gsem0, gsem1)
    wid = lax.axis_index("s") * NC + lax.axis_index("c")

    for f in range(nf):
        emb_ref = emb_refs[f]

        def stage(b0, buf, emb_ref=emb_ref):
            idx_v, rows_v, gsem = idx_bufs[buf], row_bufs[buf], gsems[buf]
            pltpu.sync_copy(cats_ref.at[f, pl.ds(b0, BBLK)], idx_v)
            pltpu.sync_copy(offs_ref.at[f, pl.ds(b0, BBLK)], off_bufs[buf])

            def fire(i, carry):
                pltpu.async_copy(emb_ref.at[idx_v.at[i]], rows_v.at[i],
                                 gsem)
                return carry

            lax.fori_loop(0, BBLK, fire, 0)

        def finish(b0, buf, emb_ref=emb_ref):
            rows_v, gsem = row_bufs[buf], gsems[buf]
            off_v = off_bufs[buf]

            def drain(i, carry):
                pltpu.make_async_copy(emb_ref.at[pl.ds(0, S)],
                                      rows_v.at[i], gsem).wait()
                return carry

            lax.fori_loop(0, BBLK, drain, 0)

            def pool_row(i, carry):
                def chunk(kk, acc):
                    a0, a1 = acc
                    ov = off_v[i, pl.ds(kk * 8, 16)]
                    for u in range(8):
                        s = kk * 8 + u
                        off = ov[u]
                        x0 = rows_v[i, s, pl.ds(off, 16)]
                        x1 = rows_v[i, s, pl.ds(off + 16, 16)]
                        a0 = a0 + _mish16(x0)
                        a1 = a1 + _mish16(x1)
                    return a0, a1

                z = jnp.zeros((16,), jnp.float32)
                a0, a1 = lax.fori_loop(0, S // 8, chunk, (z, z))
                ov = off_v[i, pl.ds(48, 16)]
                for u in range(S - 8 * (S // 8)):
                    s = 48 + u
                    off = ov[u]
                    x0 = rows_v[i, s, pl.ds(off, 16)]
                    x1 = rows_v[i, s, pl.ds(off + 16, 16)]
                    a0 = a0 + _mish16(x0)
                    a1 = a1 + _mish16(x1)
                pooled_v[i, pl.ds(0, 16)] = a0 * (1.0 / S)
                pooled_v[i, pl.ds(16, 16)] = a1 * (1.0 / S)
                return carry

            lax.fori_loop(0, BBLK, pool_row, 0)
            pltpu.async_copy(pooled_v, out_ref.at[f, pl.ds(b0, BBLK)],
                             wsem).wait()

        def sweep_pair(swp, carry):
            b0 = wid * (BBLK * SWEEPS) + swp * (2 * BBLK)
            stage(b0, 0)
            stage(b0 + BBLK, 1)
            finish(b0, 0)
            finish(b0 + BBLK, 1)
            return carry

        lax.fori_loop(0, SWEEPS // 2, sweep_pair, 0)


@functools.lru_cache(maxsize=8)
def _make_sc_pool(nf):
    return pl.kernel(
        functools.partial(_sc_body, nf),
        out_type=jax.ShapeDtypeStruct((nf, B, EMB), jnp.float32),
        mesh=plsc.VectorSubcoreMesh(core_axis_name="c", subcore_axis_name="s",
                                    num_cores=NC, num_subcores=NS),
        scratch_types=[
            pltpu.VMEM((BBLK, S), jnp.int32),
            pltpu.VMEM((BBLK, S), jnp.int32),
            pltpu.VMEM((BBLK, 64), jnp.int32),
            pltpu.VMEM((BBLK, 64), jnp.int32),
            pltpu.VMEM((BBLK, S, 128), jnp.float32),
            pltpu.VMEM((BBLK, S, 128), jnp.float32),
            pltpu.VMEM((BBLK, EMB), jnp.float32),
            pltpu.SemaphoreType.DMA,
            pltpu.SemaphoreType.DMA,
            pltpu.SemaphoreType.DMA,
        ],
        compiler_params=pltpu.CompilerParams(use_tc_tiling_on_sc=False),
    )


BC = 64


def _head_body(x_ref, w_ref, b_ref, o_ref):
    acc = b_ref[...].astype(jnp.float32) * jnp.ones((BC, 1), jnp.float32)
    for f in range(N_CAT):
        acc = acc + jax.lax.dot(x_ref[f], w_ref[f],
                                precision=lax.Precision.HIGHEST)
    o_ref[...] = acc


_head = pl.pallas_call(
    _head_body,
    grid=(B // BC,),
    in_specs=[
        pl.BlockSpec((N_CAT, BC, EMB), lambda i: (0, i, 0)),
        pl.BlockSpec((N_CAT, EMB, OUT), lambda i: (0, 0, 0)),
        pl.BlockSpec((1, OUT), lambda i: (0, 0)),
    ],
    out_specs=pl.BlockSpec((BC, OUT), lambda i: (i, 0)),
    out_shape=jax.ShapeDtypeStruct((B, OUT), jnp.float32),
)


def kernel(cat_0, cat_1, cat_2, cat_3, cat_4, cat_5, cat_6, cat_7, cat_8,
           cat_9, cat_10, cat_11, cat_12, cat_13, cat_14, cat_15, cat_16,
           cat_17, cat_18, cat_19, cat_20, cat_21, cat_22, cat_23, cat_24,
           cat_25,
           emb_0, emb_1, emb_2, emb_3, emb_4, emb_5, emb_6, emb_7, emb_8,
           emb_9, emb_10, emb_11, emb_12, emb_13, emb_14, emb_15, emb_16,
           emb_17, emb_18, emb_19, emb_20, emb_21, emb_22, emb_23, emb_24,
           emb_25,
           W, b):
    cats = [cat_0, cat_1, cat_2, cat_3, cat_4, cat_5, cat_6, cat_7, cat_8,
            cat_9, cat_10, cat_11, cat_12, cat_13, cat_14, cat_15, cat_16,
            cat_17, cat_18, cat_19, cat_20, cat_21, cat_22, cat_23, cat_24,
            cat_25]
    embs = [emb_0, emb_1, emb_2, emb_3, emb_4, emb_5, emb_6, emb_7, emb_8,
            emb_9, emb_10, emb_11, emb_12, emb_13, emb_14, emb_15, emb_16,
            emb_17, emb_18, emb_19, emb_20, emb_21, emb_22, emb_23, emb_24,
            emb_25]
    groups = [7, 7, 6, 4, 2]
    pieces = []
    f0 = 0
    for nf in groups:
        cats_g = jnp.stack(cats[f0:f0 + nf])
        cats_p = cats_g % VP
        cats_o = jnp.pad((cats_g // VP) * EMB, ((0, 0), (0, 0), (0, 14)))
        embs_t = []
        for e in embs[f0:f0 + nf]:
            embs_t.extend([jnp.transpose(e)] * 4)
        embs_rm = _make_repack(nf)(*embs_t)
        pieces.append(_make_sc_pool(nf)(cats_p, cats_o, *embs_rm))
        f0 += nf
    pooled = jnp.concatenate(pieces, axis=0)
    w3 = W.reshape(N_CAT, EMB, OUT)
    b2 = b.reshape(1, OUT)
    return _head(pooled, w3, b2)
